# baseline (speedup 1.0000x reference)
import functools

import numpy as np
import jax
import jax.numpy as jnp
from jax import lax
from jax.experimental import pallas as pl
from jax.experimental.pallas import tpu as pltpu
from jax.experimental.pallas import tpu_sc as plsc

BATCH = 32
SEQ = 2048
D_MODEL = 512
LANES = 16

NUM_CORES = 2
NUM_SUBCORES = 16
NUM_WORKERS = NUM_CORES * NUM_SUBCORES
S_PER_W = SEQ // NUM_WORKERS
VREGS_PER_CHUNK = S_PER_W * D_MODEL // LANES


def _positional_encoding_host(seq_len: int, d_model: int) -> np.ndarray:
    even_i = np.arange(0, d_model, 2, dtype=np.float64)
    denominator = np.power(10000.0, even_i / float(d_model))
    position = np.arange(seq_len, dtype=np.float64).reshape(seq_len, 1)
    pe = np.empty((seq_len, d_model), dtype=np.float32)
    pe[:, 0::2] = np.sin(position / denominator).astype(np.float32)
    pe[:, 1::2] = np.cos(position / denominator).astype(np.float32)
    return pe


NBUF = 5
PREFETCH = 3
SUB = 32
NSUB = BATCH * (S_PER_W // SUB)
SUB_VREGS = SUB * D_MODEL // LANES


def _sc_body(tokens_hbm, table_hbm, pe_hbm, out_hbm, idx_v, pe_v, rows4,
             g0, g1, g2, g3, g4, t0, t1, t2, t3, t4):
    gsems = (g0, g1, g2, g3, g4)
    ssems = (t0, t1, t2, t3, t4)
    wid = lax.axis_index("s") * NUM_CORES + lax.axis_index("c")
    s0 = wid * S_PER_W

    for b in range(BATCH):
        pltpu.sync_copy(tokens_hbm.at[pl.ds(b * SEQ + s0, S_PER_W)], idx_v.at[b])
    pltpu.sync_copy(pe_hbm.at[pl.ds(s0, S_PER_W)], pe_v)

    def fire_gather(i):
        n = i % NBUF
        b, h = divmod(i, S_PER_W // SUB)
        return pltpu.async_copy(
            table_hbm.at[idx_v.at[b, pl.ds(h * SUB, SUB)]],
            rows4.at[n], gsems[n])

    gd, sd = {}, {}
    for i in range(PREFETCH):
        gd[i] = fire_gather(i)

    for i in range(NSUB):
        n = i % NBUF
        b, h = divmod(i, S_PER_W // SUB)
        gd.pop(i).wait()

        @plsc.parallel_loop(0, SUB_VREGS, 1, unroll=8)
        def _add(k, _n=n, _h=h):
            r = k >> 5
            col = pl.multiple_of((k & 31) << 4, LANES)
            plsc.addupdate(rows4.at[_n, r, pl.ds(col, LANES)],
                           pe_v[_h * SUB + r, pl.ds(col, LANES)])

        sd[i] = pltpu.async_copy(
            rows4.at[n], out_hbm.at[b, pl.ds(s0 + h * SUB, SUB)], ssems[n])

        j = i + PREFETCH
        if j < NSUB:
            if j - NBUF >= 0:
                sd.pop(j - NBUF).wait()
            gd[j] = fire_gather(j)

    for i in sorted(sd):
        sd[i].wait()


@functools.partial(jax.jit, static_argnames=())
def kernel(tokens, table):
    pe = jnp.asarray(_positional_encoding_host(SEQ, D_MODEL))
    mesh = plsc.VectorSubcoreMesh(core_axis_name="c", subcore_axis_name="s")
    run = pl.kernel(
        _sc_body,
        out_type=jax.ShapeDtypeStruct((BATCH, SEQ, D_MODEL), jnp.float32),
        mesh=mesh,
        scratch_types=[
            pltpu.VMEM((BATCH, S_PER_W), jnp.int32),
            pltpu.VMEM((S_PER_W, D_MODEL), jnp.float32),
            pltpu.VMEM((NBUF, SUB, D_MODEL), jnp.float32),
        ] + [pltpu.SemaphoreType.DMA] * (2 * NBUF),
    )
    return run(tokens.reshape(-1), table, pe)
